# Initial kernel scaffold; baseline (speedup 1.0000x reference)
import jax
import jax.numpy as jnp
from jax.experimental import pallas as pl


def kernel(features, edge_index, edge_type, W_rel, W_self, bias):
    raise NotImplementedError("write your pallas kernel here")


# speedup vs baseline: 15.1385x; 15.1385x over previous
import functools

import jax
import jax.numpy as jnp
from jax import lax
from jax.experimental import pallas as pl
from jax.experimental.pallas import tpu as pltpu
from jax.experimental.pallas import tpu_sc as plsc

_NC = 2
_NS = 16
_NW = _NC * _NS

_K = 80


def _xw_body(f_ref, w_ref, o_ref):
    o_ref[0] = jnp.dot(f_ref[...], w_ref[0], preferred_element_type=jnp.float32)


def _combine_body(p_ref, f_ref, ws_ref, b_ref, o_ref):
    h = jnp.dot(f_ref[...], ws_ref[...], preferred_element_type=jnp.float32)
    o_ref[...] = h + p_ref[0] + p_ref[1] + b_ref[...]


def _make_sc_kernel(n_pad, d_out, chunks):
    mesh = plsc.VectorSubcoreMesh(
        core_axis_name="c", subcore_axis_name="s", num_cores=_NC, num_subcores=_NS
    )
    rows_per_tile = n_pad // _NS

    @functools.partial(
        pl.kernel,
        mesh=mesh,
        out_type=jax.ShapeDtypeStruct((_NC, n_pad, d_out), jnp.float32),
        scratch_types=[
            pltpu.VMEM((chunks, _K), jnp.int32),
            pltpu.VMEM((chunks, _K), jnp.int32),
            pltpu.VMEM((_K, d_out), jnp.float32),
            pltpu.VMEM_SHARED((n_pad, d_out), jnp.float32),
            pltpu.SemaphoreType.DMA,
        ],
    )
    def sc_kernel(xw_hbm, gidx_hbm, dst_hbm, zero_hbm, out_hbm,
                  idx_v, dst_v, rows_v, acc_sh, sem):
        c = lax.axis_index("c")
        s = lax.axis_index("s")
        w = c * _NS + s

        r0 = s * rows_per_tile
        pltpu.sync_copy(
            zero_hbm.at[pl.ds(r0, rows_per_tile)],
            acc_sh.at[pl.ds(r0, rows_per_tile)],
        )
        pltpu.sync_copy(gidx_hbm.at[w], idx_v)
        pltpu.sync_copy(dst_hbm.at[w], dst_v)
        plsc.subcore_barrier()

        def body(ci, carry):
            pltpu.async_copy(xw_hbm.at[idx_v.at[ci]], rows_v, sem).wait()
            pltpu.sync_copy(rows_v, acc_sh.at[dst_v.at[ci]], add=True)
            return carry

        lax.fori_loop(0, chunks, body, 0)
        plsc.subcore_barrier()

        pltpu.sync_copy(
            acc_sh.at[pl.ds(r0, rows_per_tile)],
            out_hbm.at[c, pl.ds(r0, rows_per_tile)],
        )

    return sc_kernel


def kernel(features, edge_index, edge_type, W_rel, W_self, bias):
    n_nodes, d_in = features.shape
    n_rel, _, d_out = W_rel.shape
    n_edges = edge_type.shape[0]

    bn = 400
    n_blocks = n_nodes // bn

    xw = pl.pallas_call(
        _xw_body,
        grid=(n_rel, n_blocks),
        in_specs=[
            pl.BlockSpec((bn, d_in), lambda r, i: (i, 0)),
            pl.BlockSpec((1, d_in, d_out), lambda r, i: (r, 0, 0)),
        ],
        out_specs=pl.BlockSpec((1, bn, d_out), lambda r, i: (r, i, 0)),
        out_shape=jax.ShapeDtypeStruct((n_rel, n_nodes, d_out), jnp.float32),
    )(features, W_rel)
    xw_flat = xw.reshape(n_rel * n_nodes, d_out)

    src = edge_index[0]
    dst = edge_index[1]
    chunks = n_edges // (_NW * _K)
    gidx = (edge_type * n_nodes + src).reshape(_NW, chunks, _K)
    dst2 = dst.reshape(_NW, chunks, _K)
    n_pad = ((n_nodes + 8 * _NS - 1) // (8 * _NS)) * (8 * _NS)
    zeros_init = jnp.zeros((n_pad, d_out), jnp.float32)

    partials = _make_sc_kernel(n_pad, d_out, chunks)(
        xw_flat, gidx, dst2, zeros_init
    )

    h = pl.pallas_call(
        _combine_body,
        grid=(n_blocks,),
        in_specs=[
            pl.BlockSpec((_NC, bn, d_out), lambda i: (0, i, 0)),
            pl.BlockSpec((bn, d_in), lambda i: (i, 0)),
            pl.BlockSpec((d_in, d_out), lambda i: (0, 0)),
            pl.BlockSpec((d_out,), lambda i: (0,)),
        ],
        out_specs=pl.BlockSpec((bn, d_out), lambda i: (i, 0)),
        out_shape=jax.ShapeDtypeStruct((n_nodes, d_out), jnp.float32),
    )(partials, features, W_self, bias)
    return h
